# baseline (speedup 1.0000x reference)
import functools

import jax
import jax.numpy as jnp
from jax import lax
from jax.experimental import pallas as pl
from jax.experimental.pallas import tpu as pltpu
from jax.experimental.pallas import tpu_sc as plsc

_N_FIELDS = 26
_FIELD_DIM = 100000
_UNIFIED_DIM = 32
_BATCH = 4096
_TOTAL = _BATCH * _N_FIELDS
_NC, _NS = 2, 16
_NW = _NC * _NS
_PER_W = _TOTAL // _NW
_CHUNK = 128
_N_CHUNK = _PER_W // _CHUNK
_SUB = _CHUNK // 16


def _build_sc_kernel():
    mesh = plsc.VectorSubcoreMesh(core_axis_name="c", subcore_axis_name="s")

    @functools.partial(
        pl.kernel,
        mesh=mesh,
        out_type=jax.ShapeDtypeStruct((_TOTAL, _UNIFIED_DIM), jnp.float32),
        scratch_types=[
            pltpu.VMEM((_N_CHUNK, _CHUNK), jnp.int32),
            pltpu.VMEM((_N_CHUNK, _CHUNK), jnp.int32),
            pltpu.VMEM((_PER_W, _UNIFIED_DIM), jnp.float32),
            pltpu.SemaphoreType.DMA,
        ],
        compiler_params=pltpu.CompilerParams(use_tc_tiling_on_sc=False),
    )
    def sc_gather(x_hbm, w_hbm, off_hbm, out_hbm, idx_v, off_v, rows_v, sem):
        wid = lax.axis_index("s") * _NC + lax.axis_index("c")
        pltpu.sync_copy(x_hbm.at[wid], idx_v)
        pltpu.sync_copy(off_hbm, off_v)
        copies = []
        for i in range(_N_CHUNK):
            for j in range(_SUB):
                sl = pl.ds(j * 16, 16)
                idx_v[i, sl] = idx_v[i, sl] + off_v[i, sl]
            c = pltpu.make_async_copy(
                w_hbm.at[idx_v.at[i]],
                rows_v.at[pl.ds(i * _CHUNK, _CHUNK)],
                sem,
            )
            c.start()
            copies.append(c)
        for c in copies:
            c.wait()
        pltpu.sync_copy(rows_v, out_hbm.at[pl.ds(wid * _PER_W, _PER_W)])

    return sc_gather


_SC_GATHER = _build_sc_kernel()


@jax.jit
def kernel(x_batch, W, embed_offsets):
    x_view = x_batch.reshape(_NW, _N_CHUNK, _CHUNK)
    off_pad = jnp.concatenate(
        [jnp.zeros((1,), jnp.int32), embed_offsets.astype(jnp.int32)]
    )
    off_pattern = jnp.tile(off_pad, _PER_W // _N_FIELDS).reshape(
        _N_CHUNK, _CHUNK
    )
    out = _SC_GATHER(x_view, W, off_pattern)
    return out.reshape(_BATCH, _N_FIELDS * _UNIFIED_DIM)
